# baseline (speedup 1.0000x reference)
import jax
import jax.numpy as jnp
from jax.experimental import pallas as pl

_B = 4096
_D = 64
_L = 4
_K = 1024
_BB = 256


def _rkmeans_block(x_ref, cbs_ref, c2_ref, logits_ref, probs_ref):
    residual = x_ref[...].astype(jnp.bfloat16)
    c2 = c2_ref[...]
    lane = jax.lax.broadcasted_iota(jnp.int32, (_BB, _K), 1)
    for l in range(_L):
        cb = cbs_ref[l]
        r32 = residual.astype(jnp.float32)
        mm = jax.lax.dot_general(
            residual, cb, (((1,), (1,)), ((), ())),
            preferred_element_type=jnp.float32)
        r2 = jnp.sum(r32 * r32, axis=1, keepdims=True)
        d2 = (r2 - 2.0 * mm) + c2[l][None, :]
        m = jnp.min(d2, axis=1, keepdims=True)
        cand = jnp.where(d2 == m, lane, _K)
        code = jnp.min(cand, axis=1, keepdims=True)
        onehot = lane == code
        probs_ref[:, l * _K:(l + 1) * _K] = onehot.astype(jnp.float32)
        logits_ref[:, l * _K:(l + 1) * _K] = jnp.where(
            onehot, jnp.float32(1.0), jnp.float32(-jnp.inf))
        if l + 1 < _L:
            g = jax.lax.dot_general(
                onehot.astype(jnp.bfloat16), cb, (((1,), (0,)), ((), ())),
                preferred_element_type=jnp.float32).astype(jnp.bfloat16)
            residual = residual - g


def kernel(x, codebooks):
    cbs = codebooks.astype(jnp.bfloat16)
    c32 = cbs.astype(jnp.float32)
    c2 = jnp.sum(c32 * c32, axis=-1)
    logits2, probs2 = pl.pallas_call(
        _rkmeans_block,
        grid=(_B // _BB,),
        in_specs=[
            pl.BlockSpec((_BB, _D), lambda i: (i, 0)),
            pl.BlockSpec((_L, _K, _D), lambda i: (0, 0, 0)),
            pl.BlockSpec((_L, _K), lambda i: (0, 0)),
        ],
        out_specs=[
            pl.BlockSpec((_BB, _L * _K), lambda i: (i, 0)),
            pl.BlockSpec((_BB, _L * _K), lambda i: (i, 0)),
        ],
        out_shape=[
            jax.ShapeDtypeStruct((_B, _L * _K), jnp.float32),
            jax.ShapeDtypeStruct((_B, _L * _K), jnp.float32),
        ],
    )(x, cbs, c2)
    return logits2.reshape(_B, _L, _K), probs2.reshape(_B, _L, _K)
